# baseline (speedup 1.0000x reference)
import math

import numpy as np
import jax
import jax.numpy as jnp
from jax import lax
from jax.experimental import pallas as pl
from jax.experimental.pallas import tpu as pltpu

RES = 32
B, C = 8, 64
N_PTS = 16384
K_CELLS = RES * RES * RES

JB = 512
LW = 512
PT = 512
GKC = 2048


def _halton_np(n, base):
    seq = np.zeros(n, dtype=np.float64)
    for i in range(n):
        f, r = 1.0, 0.0
        k = i + 1
        while k > 0:
            f /= base
            r += f * (k % base)
            k //= base
        seq[i] = r
    return seq


_HALTON2 = np.stack([_halton_np(N_PTS, 2), _halton_np(N_PTS, 3)], 0).astype(np.float32)


def _threefry2x32(k0, k1, x0, x1):
    ks0 = k0
    ks1 = k1
    ks2 = k0 ^ k1 ^ jnp.int32(0x1BD11BDA)
    ks = (ks0, ks1, ks2)
    rot = ((13, 15, 26, 6), (17, 29, 16, 24))
    x0 = x0 + ks0
    x1 = x1 + ks1
    for g in range(1, 6):
        for r in rot[(g - 1) % 2]:
            x0 = x0 + x1
            x1 = (x1 << r) | lax.shift_right_logical(x1, 32 - r)
            x1 = x1 ^ x0
        x0 = x0 + ks[g % 3]
        x1 = x1 + ks[(g + 1) % 3] + jnp.int32(g)
    return x0, x1


def _sample_body(keys_ref, l_ref, out_ref):
    i = pl.program_id(0)
    jblk = pl.program_id(1)
    k0 = keys_ref[i, 0]
    k1 = keys_ref[i, 1]
    row0 = jblk * JB
    j_iota = lax.broadcasted_iota(jnp.int32, (JB, LW), 0)
    c_iota = lax.broadcasted_iota(jnp.int32, (JB, LW), 1)
    m_base = (row0 + j_iota) * K_CELLS + c_iota
    kcn = K_CELLS // LW

    def chunk(t, carry):
        best_v, best_i = carry
        m = m_base + t * LW
        b0, b1 = _threefry2x32(k0, k1, jnp.zeros_like(m), m)
        bits = b0 ^ b1
        f = lax.bitcast_convert_type(
            lax.shift_right_logical(bits, 9) | jnp.int32(0x3F800000), jnp.float32
        ) - jnp.float32(1.0)
        u = jnp.maximum(f, jnp.float32(np.finfo(np.float32).tiny))
        g = -jnp.log(-jnp.log(u))
        v = g + l_ref[0, 0, pl.ds(t * LW, LW)][None, :]
        upd = v > best_v
        best_v = jnp.where(upd, v, best_v)
        best_i = jnp.where(upd, jnp.int32(t * LW) + c_iota, best_i)
        return best_v, best_i

    init_v = jnp.full((JB, LW), -jnp.inf, jnp.float32)
    init_i = jnp.zeros((JB, LW), jnp.int32)
    best_v, best_i = lax.fori_loop(0, kcn, chunk, (init_v, init_i))
    rmax = jnp.max(best_v, axis=1, keepdims=True)
    cand = jnp.where(best_v >= rmax, best_i, jnp.int32(K_CELLS))
    out_ref[0, 0, 0, :] = jnp.min(cand, axis=1)


def _expand_body(s_ref, cnt_ref):
    kt = pl.program_id(1)
    k_iota = kt * LW + lax.broadcasted_iota(jnp.int32, (1, LW), 1)

    def acc(t, tot):
        srow = s_ref[0, 0, pl.ds(t * LW, LW)].reshape(LW, 1)
        return tot + jnp.sum((srow <= k_iota).astype(jnp.int32), axis=0,
                             keepdims=True)

    tot = lax.fori_loop(0, N_PTS // LW, acc, jnp.zeros((1, LW), jnp.int32))
    cnt_ref[0, 0, :] = tot[0, :]


def _rank_body(cnt_ref, ind_ref):
    ptile = pl.program_id(1)
    p_iota = ptile * LW + lax.broadcasted_iota(jnp.int32, (1, LW), 1)

    def acc(t, tot):
        crow = cnt_ref[0, 0, pl.ds(t * LW, LW)].reshape(LW, 1)
        return tot + jnp.sum((crow <= p_iota).astype(jnp.int32), axis=0,
                             keepdims=True)

    tot = lax.fori_loop(0, K_CELLS // LW, acc, jnp.zeros((1, LW), jnp.int32))
    ind_ref[0, 0, :] = tot[0, :]


def _final_body(ind_ref, x_ref, wp_ref, wb_ref, br_ref, out_ref, reg_ref,
                y_ref):
    def ystep(t, _):
        xc = x_ref[0, :, pl.ds(t * GKC, GKC)]
        y_ref[:, pl.ds(t * GKC, GKC)] = jnp.dot(
            wp_ref[:, :], xc, preferred_element_type=jnp.float32,
            precision=lax.Precision.HIGHEST)
        return 0

    lax.fori_loop(0, K_CELLS // GKC, ystep, 0)
    thr = jnp.float32(math.sqrt(3.0) / RES)
    sc = jnp.float32(2.0 / RES)
    off = jnp.float32(1.0 / RES - 1.0)

    def pstep(t, _):
        ind = ind_ref[0, 0, pl.ds(t * PT, PT)]
        ind_row = ind[None, :]
        wb = jnp.dot(wb_ref[:, :], br_ref[0, :, pl.ds(t * PT, PT)],
                     preferred_element_type=jnp.float32,
                     precision=lax.Precision.HIGHEST)

        def gstep(q, acc):
            kio = q * GKC + lax.broadcasted_iota(jnp.int32, (GKC, PT), 0)
            oh = (kio == ind_row).astype(jnp.float32)
            return acc + jnp.dot(y_ref[:, pl.ds(q * GKC, GKC)], oh,
                                 preferred_element_type=jnp.float32,
                                 precision=lax.Precision.HIGHEST)

        g = lax.fori_loop(0, K_CELLS // GKC, gstep,
                          jnp.zeros((8, PT), jnp.float32))
        o = g + wb
        ox, oy, oz = o[0, :], o[1, :], o[2, :]
        nrm = jnp.sqrt(ox * ox + oy * oy + oz * oz)
        reg_ref[0, 0, pl.ds(t * PT, PT)] = jnp.maximum(nrm - thr, 0.0)
        cx = lax.shift_right_logical(ind, 10)
        cy = lax.shift_right_logical(ind, 5) & 31
        cz = ind & 31
        out_ref[0, 0, pl.ds(t * PT, PT)] = ox + cx.astype(jnp.float32) * sc + off
        out_ref[0, 1, pl.ds(t * PT, PT)] = oy + cy.astype(jnp.float32) * sc + off
        out_ref[0, 2, pl.ds(t * PT, PT)] = oz + cz.astype(jnp.float32) * sc + off
        return 0

    lax.fori_loop(0, N_PTS // PT, pstep, 0)


def kernel(x, dens, W_gen, b_gen, n):
    b = x.shape[0]
    x_flat = x.reshape(b, C, K_CELLS)

    key = jax.random.key(42)
    keys = jnp.stack(
        [jax.random.key_data(jax.random.fold_in(key, i)) for i in range(b)], 0
    ).astype(jnp.int32)

    d = dens.reshape(b, K_CELLS)
    dsum = jnp.sum(d, axis=1, keepdims=True)
    d = jnp.where(dsum < 1e-12, jnp.ones_like(d), d)
    logits = jnp.log(d + jnp.float32(1e-30)).reshape(b, 1, K_CELLS)

    samples = pl.pallas_call(
        _sample_body,
        grid_spec=pltpu.PrefetchScalarGridSpec(
            num_scalar_prefetch=1,
            grid=(b, N_PTS // JB),
            in_specs=[
                pl.BlockSpec((1, 1, K_CELLS), lambda i, j, keys: (i, 0, 0))
            ],
            out_specs=pl.BlockSpec((1, 1, 1, JB), lambda i, j, keys: (i, j, 0, 0)),
        ),
        out_shape=jax.ShapeDtypeStruct((b, N_PTS // JB, 1, JB), jnp.int32),
    )(keys, logits)
    samples = samples.reshape(b, 1, N_PTS)

    cnt = pl.pallas_call(
        _expand_body,
        grid=(b, K_CELLS // LW),
        in_specs=[pl.BlockSpec((1, 1, N_PTS), lambda i, j: (i, 0, 0))],
        out_specs=pl.BlockSpec((1, 1, LW), lambda i, j: (i, 0, j)),
        out_shape=jax.ShapeDtypeStruct((b, 1, K_CELLS), jnp.int32),
    )(samples)

    ind = pl.pallas_call(
        _rank_body,
        grid=(b, N_PTS // LW),
        in_specs=[pl.BlockSpec((1, 1, K_CELLS), lambda i, j: (i, 0, 0))],
        out_specs=pl.BlockSpec((1, 1, LW), lambda i, j: (i, 0, j)),
        out_shape=jax.ShapeDtypeStruct((b, 1, N_PTS), jnp.int32),
    )(cnt)

    Wp = jnp.zeros((8, C), jnp.float32).at[:3, :].set(W_gen[:, :C])
    Wb = (jnp.zeros((8, 8), jnp.float32)
          .at[:3, :2].set(W_gen[:, C:C + 2])
          .at[:3, 2].set(b_gen))
    ratio = jnp.float32(2.0)
    b_rnd = (jnp.asarray(_HALTON2)
             + jnp.zeros((), jnp.float32) * n) * ratio - ratio / 2.0
    BR = (jnp.zeros((1, 8, N_PTS), jnp.float32)
          .at[0, :2, :].set(b_rnd)
          .at[0, 2, :].set(1.0))

    outs, regs = pl.pallas_call(
        _final_body,
        grid=(b,),
        in_specs=[
            pl.BlockSpec((1, 1, N_PTS), lambda i: (i, 0, 0)),
            pl.BlockSpec((1, C, K_CELLS), lambda i: (i, 0, 0)),
            pl.BlockSpec((8, C), lambda i: (0, 0)),
            pl.BlockSpec((8, 8), lambda i: (0, 0)),
            pl.BlockSpec((1, 8, N_PTS), lambda i: (0, 0, 0)),
        ],
        out_specs=[
            pl.BlockSpec((1, 3, N_PTS), lambda i: (i, 0, 0)),
            pl.BlockSpec((1, 1, N_PTS), lambda i: (i, 0, 0)),
        ],
        out_shape=[
            jax.ShapeDtypeStruct((b, 3, N_PTS), jnp.float32),
            jax.ShapeDtypeStruct((b, 1, N_PTS), jnp.float32),
        ],
        scratch_shapes=[pltpu.VMEM((8, K_CELLS), jnp.float32)],
    )(ind, x_flat, Wp, Wb, BR)
    return outs, regs.reshape(b, N_PTS)
